# baseline (speedup 1.0000x reference)
import functools

import jax
import jax.numpy as jnp
from jax import lax
from jax.experimental import pallas as pl
from jax.experimental.pallas import tpu as pltpu
from jax.experimental.pallas import tpu_sc as plsc

NC = 2
NS = 16
NW = NC * NS
LANES = 16
GATHER = 100
RPC = 4


def _build(batch, seq_len, d_model):
    spw = batch // (NW * RPC)
    hpr = seq_len // GATHER
    kd = d_model // LANES
    mesh = plsc.VectorSubcoreMesh(core_axis_name="c", subcore_axis_name="s")

    @functools.partial(
        pl.kernel,
        out_type=jax.ShapeDtypeStruct((batch, seq_len, d_model), jnp.float32),
        mesh=mesh,
        compiler_params=pltpu.CompilerParams(use_tc_tiling_on_sc=False),
        scratch_types=[
            pltpu.VMEM((2, RPC * hpr, GATHER), jnp.int32),
            pltpu.VMEM((RPC, seq_len, d_model), jnp.float32),
            pltpu.VMEM((RPC, seq_len, d_model), jnp.float32),
            pltpu.VMEM((seq_len, d_model), jnp.float32),
            pltpu.SemaphoreType.DMA,
            pltpu.SemaphoreType.DMA,
            pltpu.SemaphoreType.DMA,
            pltpu.SemaphoreType.DMA,
        ],
    )
    def emb(tok_hbm, table_hbm, pos_hbm, out_hbm,
            idx_v, rows0, rows1, pos_v, g0, g1, o0, o1):
        rows = (rows0, rows1)
        gsem = (g0, g1)
        wid = lax.axis_index("s") * NC + lax.axis_index("c")
        base = wid * spw
        pltpu.sync_copy(pos_hbm, pos_v)

        def fire_gathers(g, b):
            pltpu.sync_copy(
                tok_hbm.at[pl.ds((base + g) * RPC * hpr, RPC * hpr)],
                idx_v.at[b])
            for r in range(RPC):
                for h in range(hpr):
                    pltpu.async_copy(
                        table_hbm.at[idx_v.at[b, r * hpr + h]],
                        rows[b].at[r, pl.ds(h * GATHER, GATHER)],
                        gsem[b],
                    )

        def drain(sem, b):
            pltpu.make_async_copy(
                out_hbm.at[pl.ds(0, RPC)], rows[b], sem).wait()

        def add_pos(b):
            def body(i, _):
                for rr in range(4):
                    row = i * 4 + rr
                    for k in range(kd):
                        pv = pos_v[row, pl.ds(k * LANES, LANES)]
                        for r in range(RPC):
                            plsc.addupdate(
                                rows[b].at[r, row, pl.ds(k * LANES, LANES)],
                                pv,
                            )
                return 0
            lax.fori_loop(0, seq_len // 4, body, 0)

        fire_gathers(0, 0)

        def pair(p, _):
            g_even = p * 2
            g_odd = g_even + 1

            @pl.when(p >= 1)
            def _():
                drain(o1, 1)
            fire_gathers(g_odd, 1)
            drain(g0, 0)
            add_pos(0)
            pltpu.async_copy(
                rows0, out_hbm.at[pl.ds((base + g_even) * RPC, RPC)], o0)

            @pl.when(p < spw // 2 - 1)
            def _():
                drain(o0, 0)
                fire_gathers(g_even + 2, 0)
            drain(g1, 1)
            add_pos(1)
            pltpu.async_copy(
                rows1, out_hbm.at[pl.ds((base + g_odd) * RPC, RPC)], o1)
            return 0

        lax.fori_loop(0, spw // 2, pair, 0)
        drain(o0, 0)
        drain(o1, 1)

    return emb


def kernel(tokens, token_table, position_table):
    b, s = tokens.shape
    _, d_model = token_table.shape
    emb = _build(b, s, d_model)
    flat_tok = tokens.reshape(b * s // GATHER, GATHER).astype(jnp.int32)
    return emb(flat_tok, token_table, position_table)
